# baseline (speedup 1.0000x reference)
import jax
import jax.numpy as jnp
from jax.experimental import pallas as pl


_N = 4096
_TILE = 512
_K = _N // 2
_WEIGHT = 3.0
_B = 4
_MXU_TILES = 6


def _opaque_abs(x):
    xi = jax.lax.bitcast_convert_type(x, jnp.int32)
    return jax.lax.bitcast_convert_type(
        jnp.bitwise_and(xi, jnp.int32(0x7FFFFFFF)), jnp.float32)


def _chamfer_kernel(pred_ref, gtt_ref, out_ref):
    rows = []
    for b in range(_B):
        p = pred_ref[b]
        gT = gtt_ref[b]
        np2c = jnp.sum(p * p, axis=1, keepdims=True)
        ng2r = jnp.sum(gT * gT, axis=0, keepdims=True)
        gTm2 = -2.0 * gT

        colmin_m = jnp.full((1, _N), jnp.inf, dtype=jnp.float32)
        colmin_v = jnp.full((1, _N), jnp.inf, dtype=jnp.float32)
        d1_pieces = []
        for i in range(_N // _TILE):
            sl = slice(i * _TILE, (i + 1) * _TILE)
            if i < _MXU_TILES:
                cross = jnp.dot(p[sl, :], gTm2,
                                preferred_element_type=jnp.float32)
                e = cross + np2c[sl, :]
                colmin_m = jnp.minimum(
                    colmin_m, jnp.min(e, axis=0, keepdims=True))
                e4 = e + ng2r
            else:
                t0 = _opaque_abs(p[sl, 0:1] - gT[0:1, :])
                t1 = _opaque_abs(p[sl, 1:2] - gT[1:2, :])
                t2 = _opaque_abs(p[sl, 2:3] - gT[2:3, :])
                e4 = t0 * t0 + t1 * t1 + t2 * t2
                colmin_v = jnp.minimum(
                    colmin_v, jnp.min(e4, axis=0, keepdims=True))
            rm = jnp.min(e4, axis=1, keepdims=True)
            d1_pieces.append(jax.lax.transpose(rm, (1, 0)))

        d1sq = jnp.maximum(jnp.concatenate(d1_pieces, axis=1), 1e-12)
        d2sq = jnp.maximum(jnp.minimum(colmin_m + ng2r, colmin_v), 1e-12)
        rows.append(jnp.sqrt(d1sq))
        rows.append(jnp.sqrt(d2sq))

    D = jnp.concatenate(rows, axis=0)
    Db = jax.lax.bitcast_convert_type(D, jnp.int32)

    def body(i, m):
        cand = m | jnp.left_shift(jnp.int32(1), jnp.int32(30) - i)
        cnt = jnp.sum(jnp.where(Db >= cand, jnp.int32(1), jnp.int32(0)),
                      axis=1, keepdims=True)
        return jnp.where(cnt >= _K, cand, m)

    m = jax.lax.fori_loop(0, 31, body, jnp.zeros((2 * _B, 1), jnp.int32))
    t = jax.lax.bitcast_convert_type(m, jnp.float32)
    gt_mask = Db > m
    cnt_gt = jnp.sum(jnp.where(gt_mask, jnp.int32(1), jnp.int32(0)),
                     axis=1, keepdims=True)
    sum_gt = jnp.sum(jnp.where(gt_mask, D, jnp.float32(0.0)),
                     axis=1, keepdims=True)
    w = (sum_gt + (jnp.int32(_K) - cnt_gt).astype(jnp.float32) * t) / _K

    s_means = jnp.sum(D, axis=1, keepdims=True) / jnp.float32(_N)
    total = jnp.sum(s_means + _WEIGHT * w)
    out_ref[:, :] = jnp.full((1, 1), 1.0 / _B, jnp.float32) * total


def kernel(pred_pointclouds, gt_pointclouds):
    gtT = jnp.transpose(gt_pointclouds, (0, 2, 1))

    out = pl.pallas_call(
        _chamfer_kernel,
        out_shape=jax.ShapeDtypeStruct((1, 1), jnp.float32),
    )(pred_pointclouds, gtT)
    return out[0, 0]
